# baseline (speedup 1.0000x reference)
import jax
import jax.numpy as jnp
from jax import lax
from jax.experimental import pallas as pl
from jax.experimental.pallas import tpu as pltpu
from jax.experimental.pallas import tpu_sc as plsc

D = 64
B = 16384
N2 = 8192
NC = 2
NS = 16
NW = NC * NS
BPW = (B - N2) // NW
NGRP = BPW // 16
K = 16


def _nrsqrt(s):
    i = plsc.bitcast(s, jnp.int32)
    i = jnp.int32(0x5F3759DF) - lax.shift_right_logical(i, jnp.int32(1))
    y = plsc.bitcast(i, jnp.float32)
    for _ in range(3):
        y = y * (jnp.float32(1.5) - jnp.float32(0.5) * s * y * y)
    return y


def _sc_body(weight_hbm, idx_hbm, ia_hbm, out_hbm,
             idx_v, ia_v, rows_v, a_v, out_v, sem, sem_a):
    wid = lax.axis_index("s") * NC + lax.axis_index("c")

    pltpu.sync_copy(idx_hbm.at[pl.ds(wid * 2, 2)], idx_v)
    pltpu.sync_copy(ia_hbm, ia_v)

    ia = ia_v[pl.ds(0, 16)][0]
    handles = [pltpu.async_copy(weight_hbm.at[pl.ds(ia, 1)], a_v, sem_a)]
    for j in range(2):
        for k in range(8):
            v = idx_v[j, pl.ds(k * 16, 16)]
            for l in range(16):
                i = j * 128 + k * 16 + l
                handles.append(pltpu.async_copy(
                    weight_hbm.at[pl.ds(v[l], 1)],
                    rows_v.at[pl.ds(i, 1)], sem))
    for h in handles:
        h.wait()

    a_regs = [a_v[0, pl.ds(k * 16, 16)] for k in range(D // 16)]
    sa = jnp.float32(0)
    for k in range(D // 16):
        sq = a_regs[k] * a_regs[k]
        for l in range(16):
            sa = sa + sq[l]
    sa = jnp.maximum(sa, jnp.float32(1e-16))

    lanes = lax.iota(jnp.int32, 16)

    def group(g, carry):
        row_idx = g * 16 + lanes
        acc_dot = jnp.zeros((16,), jnp.float32)
        acc_sq = jnp.zeros((16,), jnp.float32)
        for d in range(D):
            col = jnp.full((16,), d, jnp.int32)
            vals = plsc.load_gather(rows_v, [row_idx, col])
            a_d = a_regs[d // 16][d % 16]
            acc_dot = acc_dot + a_d * vals
            acc_sq = acc_sq + vals * vals
        r = _nrsqrt(sa * jnp.maximum(acc_sq, jnp.float32(1e-16)))
        out_v[pl.ds(g * 16, 16)] = acc_dot * r
        return carry

    lax.fori_loop(0, NGRP, group, None)

    pltpu.sync_copy(out_v, out_hbm.at[pl.ds(wid * BPW, BPW)])


def _tc_body(idx_ref, w_ref, o_ref, rows_v, a_v, sem_a, *sems):
    pltpu.make_async_copy(w_ref.at[pl.ds(idx_ref[N2], 1)], a_v, sem_a).start()

    def outer(t, carry):
        for k in range(K):
            @pl.when(t > 0)
            def _w():
                pltpu.make_async_copy(w_ref.at[pl.ds(0, 1)],
                                      rows_v.at[pl.ds(0, 1)], sems[k]).wait()
            i = t * K + k
            pltpu.make_async_copy(w_ref.at[pl.ds(idx_ref[i], 1)],
                                  rows_v.at[pl.ds(i, 1)], sems[k]).start()
        return carry

    lax.fori_loop(0, N2 // K, outer, None)
    for k in range(K):
        pltpu.make_async_copy(w_ref.at[pl.ds(0, 1)],
                              rows_v.at[pl.ds(0, 1)], sems[k]).wait()
    pltpu.make_async_copy(w_ref.at[pl.ds(0, 1)], a_v, sem_a).wait()

    w = rows_v[...]
    a = a_v[0, :]
    p = jnp.dot(w, a, preferred_element_type=jnp.float32)
    q = jnp.sum(w * w, axis=1)
    sa = jnp.maximum(jnp.sum(a * a), jnp.float32(1e-16))
    o_ref[...] = p * lax.rsqrt(sa * jnp.maximum(q, jnp.float32(1e-16)))


def kernel(x, weight):
    xi = x[:, 1].astype(jnp.int32)
    ia = x[0, 0].astype(jnp.int32)
    idx_sc = xi[N2:].reshape(NW * 2, 128)
    ia16 = jnp.broadcast_to(ia[None], (16,))
    idx_tc = jnp.concatenate([xi[:N2], jnp.broadcast_to(ia[None], (8,))])

    out_sc = pl.kernel(
        _sc_body,
        out_type=jax.ShapeDtypeStruct((B - N2,), jnp.float32),
        mesh=plsc.VectorSubcoreMesh(core_axis_name="c", subcore_axis_name="s",
                                    num_cores=NC, num_subcores=NS),
        compiler_params=pltpu.CompilerParams(needs_layout_passes=False),
        scratch_types=[
            pltpu.VMEM((2, 128), jnp.int32),
            pltpu.VMEM((16,), jnp.int32),
            pltpu.VMEM((BPW, D), jnp.float32),
            pltpu.VMEM((1, D), jnp.float32),
            pltpu.VMEM((BPW,), jnp.float32),
            pltpu.SemaphoreType.DMA,
            pltpu.SemaphoreType.DMA,
        ],
    )(weight, idx_sc, ia16)

    out_tc = pl.pallas_call(
        _tc_body,
        grid_spec=pltpu.PrefetchScalarGridSpec(
            num_scalar_prefetch=1,
            grid=(1,),
            in_specs=[pl.BlockSpec(memory_space=pl.ANY)],
            out_specs=pl.BlockSpec((N2,), lambda i, s: (0,)),
            scratch_shapes=[pltpu.VMEM((N2, D), jnp.float32),
                            pltpu.VMEM((1, D), jnp.float32),
                            pltpu.SemaphoreType.DMA]
                           + [pltpu.SemaphoreType.DMA] * K,
        ),
        out_shape=jax.ShapeDtypeStruct((N2,), jnp.float32),
    )(idx_tc, weight)

    return jnp.concatenate([out_tc, out_sc])


# speedup vs baseline: 1.8443x; 1.8443x over previous
import jax
import jax.numpy as jnp
from jax import lax
from jax.experimental import pallas as pl
from jax.experimental.pallas import tpu as pltpu
from jax.experimental.pallas import tpu_sc as plsc

D = 64
B = 16384
NC = 2
NS = 16
NW = NC * NS
BPW = B // NW
NGRP = BPW // 16
NSEM = 4


def _nrsqrt(s):
    i = plsc.bitcast(s, jnp.int32)
    i = jnp.int32(0x5F3759DF) - lax.shift_right_logical(i, jnp.int32(1))
    y = plsc.bitcast(i, jnp.float32)
    for _ in range(3):
        y = y * (jnp.float32(1.5) - jnp.float32(0.5) * s * y * y)
    return y


def _sc_body(weight_hbm, idx_hbm, ia_hbm, out_hbm,
             idx_v, ia_v, rows_v, a_v, out_v,
             sem0, sem1, sem2, sem3, sem_a):
    wid = lax.axis_index("s") * NC + lax.axis_index("c")
    sems = (sem0, sem1, sem2, sem3)

    pltpu.sync_copy(idx_hbm.at[pl.ds(wid * 4, 4)], idx_v)
    pltpu.sync_copy(ia_hbm, ia_v)

    ia = ia_v[pl.ds(0, 16)][0]
    h_a = pltpu.async_copy(weight_hbm.at[pl.ds(ia, 1)], a_v, sem_a)

    def fire(js):
        handles = []
        for j in js:
            for k in range(8):
                v = idx_v[j, pl.ds(k * 16, 16)]
                for l in range(16):
                    i = j * 128 + k * 16 + l
                    handles.append(pltpu.async_copy(
                        weight_hbm.at[pl.ds(v[l], 1)],
                        rows_v.at[pl.ds(i, 1)], sems[i % NSEM]))
        return handles

    for h in fire((0, 1)):
        h.wait()
    batch2 = fire((2, 3))
    h_a.wait()

    a_regs = [a_v[0, pl.ds(k * 16, 16)] for k in range(D // 16)]
    sa = jnp.float32(0)
    for k in range(D // 16):
        sq = a_regs[k] * a_regs[k]
        for l in range(16):
            sa = sa + sq[l]
    sa = jnp.maximum(sa, jnp.float32(1e-16))

    lanes = lax.iota(jnp.int32, 16)

    def group(g, carry):
        row_idx = g * 16 + lanes
        acc_dot = jnp.zeros((16,), jnp.float32)
        acc_sq = jnp.zeros((16,), jnp.float32)
        for d in range(D):
            col = jnp.full((16,), d, jnp.int32)
            vals = plsc.load_gather(rows_v, [row_idx, col])
            a_d = a_regs[d // 16][d % 16]
            acc_dot = acc_dot + a_d * vals
            acc_sq = acc_sq + vals * vals
        r = _nrsqrt(sa * jnp.maximum(acc_sq, jnp.float32(1e-16)))
        out_v[pl.ds(g * 16, 16)] = acc_dot * r
        return carry

    lax.fori_loop(0, NGRP // 2, group, None)
    for h in batch2:
        h.wait()
    lax.fori_loop(NGRP // 2, NGRP, group, None)

    pltpu.sync_copy(out_v, out_hbm.at[pl.ds(wid * BPW, BPW)])


def kernel(x, weight):
    idx = x[:, 1].astype(jnp.int32).reshape(NW * 4, 128)
    ia = jnp.broadcast_to(x[0, 0].astype(jnp.int32)[None], (16,))
    return pl.kernel(
        _sc_body,
        out_type=jax.ShapeDtypeStruct((B,), jnp.float32),
        mesh=plsc.VectorSubcoreMesh(core_axis_name="c", subcore_axis_name="s",
                                    num_cores=NC, num_subcores=NS),
        compiler_params=pltpu.CompilerParams(needs_layout_passes=False),
        scratch_types=[
            pltpu.VMEM((4, 128), jnp.int32),
            pltpu.VMEM((16,), jnp.int32),
            pltpu.VMEM((BPW, D), jnp.float32),
            pltpu.VMEM((1, D), jnp.float32),
            pltpu.VMEM((BPW,), jnp.float32),
            pltpu.SemaphoreType.DMA,
            pltpu.SemaphoreType.DMA,
            pltpu.SemaphoreType.DMA,
            pltpu.SemaphoreType.DMA,
            pltpu.SemaphoreType.DMA,
        ],
    )(weight, idx, ia)
